# Initial kernel scaffold; baseline (speedup 1.0000x reference)
import jax
import jax.numpy as jnp
from jax.experimental import pallas as pl


def kernel(input_ids, text_w, audio_w, special_w, phoneme_w, reserved_w):
    raise NotImplementedError("write your pallas kernel here")


# speedup vs baseline: 6.2183x; 6.2183x over previous
import functools

import jax
import jax.numpy as jnp
import numpy as np
from jax import lax
from jax.experimental import pallas as pl
from jax.experimental.pallas import tpu as pltpu
from jax.experimental.pallas import tpu_sc as plsc

D_MODEL = 1024
TABLE_STARTS = (0, 100000, 110000, 111000, 116000)

B_TOTAL = 4 * 8192
NUM_WORKERS = 32
CHUNK = B_TOTAL // NUM_WORKERS
G = 16
NVREG = CHUNK // 16
LIST_CAP = CHUNK + G


def _gather16(v, idx):
    return v.at[idx].get(mode="promise_in_bounds")


def _lt0(x):
    return jnp.where(x < 0, 1, 0)


def _ge0(x):
    return jnp.where(x >= 0, 1, 0)


def _blend(b, a, m01):
    return b + (a - b) * m01


def _prefix16(v, lane):
    for k in (1, 2, 4, 8):
        shifted = _gather16(v, jnp.maximum(lane - k, 0))
        v = v + shifted * _ge0(lane - k)
    return v


def _invert16(m, lane):
    srcs = jnp.zeros((16,), jnp.int32)
    for l in range(16):
        srcs = srcs + jnp.where(lane == m[l], l, 0)
    return srcs


def _sc_embed(ids, text_w, audio_w, special_w, phoneme_w, reserved_w):
    mesh = plsc.VectorSubcoreMesh(core_axis_name="c", subcore_axis_name="s")

    list_scratch = [pltpu.VMEM((LIST_CAP,), jnp.int32) for _ in range(10)]

    @functools.partial(
        pl.kernel,
        mesh=mesh,
        out_type=jax.ShapeDtypeStruct((B_TOTAL, D_MODEL), jnp.float32),
        scratch_types=[
            pltpu.VMEM((CHUNK,), jnp.int32),
            *list_scratch,
            pltpu.VMEM((G, D_MODEL), jnp.float32),
            pltpu.SemaphoreType.DMA,
            pltpu.SemaphoreType.DMA,
        ],
    )
    def k(ids_hbm, t0, t1, t2, t3, t4, out_hbm, ids_v, *rest):
        tables = (t0, t1, t2, t3, t4)
        idx_list = rest[0:5]
        pos_list = rest[5:10]
        cbuf = rest[10]
        gsem, ssem = rest[11], rest[12]

        wid = lax.axis_index("s") * 2 + lax.axis_index("c")
        base = wid * CHUNK
        pltpu.sync_copy(ids_hbm.at[pl.ds(base, CHUNK)], ids_v)

        lane = lax.iota(jnp.int32, 16)
        zeros16 = jnp.zeros((16,), jnp.int32)

        def compact_body(i, carry):
            offs, cnts, win_i, win_p = carry
            start = pl.multiple_of(i * 16, 16)
            ids_vec = ids_v[pl.ds(start, 16)]
            pos_vec = base + i * 16 + lane
            ge1 = _ge0(ids_vec - TABLE_STARTS[1])
            ge2 = _ge0(ids_vec - TABLE_STARTS[2])
            ge3 = _ge0(ids_vec - TABLE_STARTS[3])
            ge4 = _ge0(ids_vec - TABLE_STARTS[4])
            tid = ge1 + ge2 + ge3 + ge4
            local = (ids_vec
                     - TABLE_STARTS[1] * ge1
                     - (TABLE_STARTS[2] - TABLE_STARTS[1]) * ge2
                     - (TABLE_STARTS[3] - TABLE_STARTS[2]) * ge3
                     - (TABLE_STARTS[4] - TABLE_STARTS[3]) * ge4)
            fives = tid * 5
            csum_all = _prefix16(jnp.int32(1) << fives, lane)
            rank = ((csum_all >> fives) & 31) - 1
            last = csum_all[15]
            cnt = [(last >> (5 * t)) & 31 for t in range(5)]
            q = [np.int32(0)]
            for t in range(4):
                q.append(q[t] + cnt[t])
            q_sel = (q[0]
                     + (q[1] - q[0]) * ge1
                     + (q[2] - q[1]) * ge2
                     + (q[3] - q[2]) * ge3
                     + (q[4] - q[3]) * ge4)
            m = q_sel + rank
            srcs = _invert16(m, lane)
            sorted_i = _gather16(local, srcs)
            sorted_p = _gather16(pos_vec, srcs)

            new_offs, new_cnts, new_wi, new_wp = [], [], [], []
            for t in range(5):
                cv = cnts[t]
                total_v = cv + cnt[t]
                lane_mc = lane - cv
                lo01 = _ge0(lane_mc) * _lt0(lane_mc - cnt[t])
                lo_idx = (lane_mc + q[t]) & 15
                wi_m = _blend(win_i[t], _gather16(sorted_i, lo_idx), lo01)
                wp_m = _blend(win_p[t], _gather16(sorted_p, lo_idx), lo01)
                hi01 = _lt0(lane + 16 - total_v)
                hi_idx = (lane_mc + 16 + q[t]) & 15
                hi_i = _gather16(sorted_i, hi_idx) * hi01
                hi_p = _gather16(sorted_p, hi_idx) * hi01

                boff = pl.multiple_of(offs[t], 16)
                idx_list[t][pl.ds(boff, 16)] = wi_m
                pos_list[t][pl.ds(boff, 16)] = wp_m

                f01 = total_v >> 4
                total_s = (total_v + lane)[0]
                new_offs.append(offs[t] + (total_s >> 4) * 16)
                new_cnts.append(total_v - f01 * 16)
                new_wi.append(_blend(wi_m, hi_i, f01))
                new_wp.append(_blend(wp_m, hi_p, f01))
            return (tuple(new_offs), tuple(new_cnts),
                    tuple(new_wi), tuple(new_wp))

        zero5 = (np.int32(0),) * 5
        zvec5 = (zeros16,) * 5
        offs, cnts, win_i, win_p = lax.fori_loop(
            0, NVREG, compact_body, (zero5, zvec5, zvec5, zvec5))

        for t in range(5):
            dup_i = _gather16(win_i[t], zeros16)
            dup_p = _gather16(win_p[t], zeros16)
            tail01 = _lt0(lane - cnts[t])
            boff = pl.multiple_of(offs[t], 16)
            idx_list[t][pl.ds(boff, 16)] = _blend(dup_i, win_i[t], tail01)
            pos_list[t][pl.ds(boff, 16)] = _blend(dup_p, win_p[t], tail01)

        for t in range(5):
            nblk = (offs[t] + (cnts[t] + lane)[0] + (G - 1)) // G

            def blk_body(j, carry, t=t):
                boff = pl.multiple_of(j * G, 16)
                iv = idx_list[t][pl.ds(boff, G)]
                pv = pos_list[t][pl.ds(boff, G)]
                pltpu.async_copy(tables[t].at[iv], cbuf, gsem).wait()
                pltpu.async_copy(cbuf, out_hbm.at[pv], ssem).wait()
                return carry

            lax.fori_loop(0, nblk, blk_body, np.int32(0))

    return k(ids, text_w, audio_w, special_w, phoneme_w, reserved_w)


def kernel(input_ids, text_w, audio_w, special_w, phoneme_w, reserved_w):
    ids = input_ids.reshape(-1).astype(jnp.int32)
    out = _sc_embed(ids, text_w, audio_w, special_w, phoneme_w, reserved_w)
    return out.reshape(input_ids.shape + (D_MODEL,))
